# baseline (speedup 1.0000x reference)
import functools

import jax
import jax.numpy as jnp
from jax import lax
from jax.experimental import pallas as pl
from jax.experimental.pallas import tpu as pltpu
from jax.experimental.pallas import tpu_sc as plsc

BATCH = 1024
MAX_LEN = 200
D_MODEL = 128
VOCAB = 128

NC, NS = 2, 16
NW = NC * NS
SENT_SC = 512
SENT_TC = BATCH - SENT_SC
TOK_SC = SENT_SC * MAX_LEN
TPW = TOK_SC // NW
CH = 64
NCHUNK = TPW // CH
NBUF = 10
NGROUP = NCHUNK // NBUF
SB = 8
NTCBLK = SENT_TC // SB


def _pe():
    pos = jnp.arange(MAX_LEN, dtype=jnp.float32)[:, None]
    i = jnp.arange(0, D_MODEL, 2, dtype=jnp.float32)
    div = jnp.exp(-(jnp.log(10000.0) / D_MODEL) * i)
    ang = pos * div[None, :]
    pe = jnp.zeros((MAX_LEN, D_MODEL), dtype=jnp.float32)
    pe = pe.at[:, 0::2].set(jnp.sin(ang))
    pe = pe.at[:, 1::2].set(jnp.cos(ang))
    return pe


def _fuse_body(table_ref, pe_ref, idx_ref, fused_ref, idx2_ref):
    fused_ref[...] = pe_ref[...][:, None, :] + table_ref[...][None, :, :]
    t = lax.broadcasted_iota(jnp.int32, (NW, TPW), 1)
    idx2_ref[...] = idx_ref[...] + lax.rem(t, MAX_LEN) * VOCAB


_fuse = pl.pallas_call(
    _fuse_body,
    out_shape=(
        jax.ShapeDtypeStruct((MAX_LEN, VOCAB, D_MODEL), jnp.float32),
        jax.ShapeDtypeStruct((NW, TPW), jnp.int32),
    ),
)


def _tc_half_body(idx_ref, table_ref, pe_ref, out_ref):
    onehot = (
        idx_ref[...][..., None]
        == lax.broadcasted_iota(jnp.int32, (SB, MAX_LEN, VOCAB), 2)
    ).astype(jnp.float32)
    rows = lax.dot_general(
        onehot,
        table_ref[...],
        (((2,), (0,)), ((), ())),
        preferred_element_type=jnp.float32,
    )
    out_ref[...] = rows + pe_ref[...][None, :, :]


_tc_half = pl.pallas_call(
    _tc_half_body,
    out_shape=jax.ShapeDtypeStruct((SENT_TC, MAX_LEN, D_MODEL), jnp.float32),
    grid=(NTCBLK,),
    in_specs=[
        pl.BlockSpec((SB, MAX_LEN), lambda s: (s, 0)),
        pl.BlockSpec((VOCAB, D_MODEL), lambda s: (0, 0)),
        pl.BlockSpec((MAX_LEN, D_MODEL), lambda s: (0, 0)),
    ],
    out_specs=pl.BlockSpec((SB, MAX_LEN, D_MODEL), lambda s: (s, 0, 0)),
)

_mesh = plsc.VectorSubcoreMesh(core_axis_name="c", subcore_axis_name="s")


@functools.partial(
    pl.kernel,
    out_type=jax.ShapeDtypeStruct((TOK_SC, D_MODEL), jnp.float32),
    mesh=_mesh,
    scratch_types=[
        pltpu.VMEM((TPW,), jnp.int32),
        pltpu.VMEM((NBUF, CH, D_MODEL), jnp.float32),
        pltpu.SemaphoreType.DMA((NBUF,)),
        pltpu.SemaphoreType.DMA((NBUF,)),
    ],
)
def _sc_gather(fused_hbm, idx_hbm, out_hbm, idx_v, rows_v, gsem, ssem):
    wid = lax.axis_index("s") * NC + lax.axis_index("c")
    pltpu.sync_copy(idx_hbm.at[wid], idx_v)
    base = wid * TPW

    def out_slice(c):
        off = pl.multiple_of(base + c * CH, 8)
        return out_hbm.at[pl.ds(off, CH)]

    def fire_gather(c, b):
        return pltpu.async_copy(
            fused_hbm.at[idx_v.at[pl.ds(c * CH, CH)]], rows_v.at[b], gsem.at[b]
        )

    gds = [fire_gather(b, b) for b in range(NBUF)]
    for b in range(NBUF):
        gds[b].wait()
        pltpu.async_copy(rows_v.at[b], out_slice(b), ssem.at[b])

    @pl.loop(1, NGROUP)
    def group_body(g):
        c0 = g * NBUF
        gds = []
        for b in range(NBUF):
            pltpu.make_async_copy(rows_v.at[b], out_slice(c0 + b), ssem.at[b]).wait()
            gds.append(fire_gather(c0 + b, b))
        for b in range(NBUF):
            gds[b].wait()
            pltpu.async_copy(rows_v.at[b], out_slice(c0 + b), ssem.at[b])

    for b in range(NBUF):
        pltpu.make_async_copy(
            rows_v.at[b], out_slice((NGROUP - 1) * NBUF + b), ssem.at[b]
        ).wait()


def kernel(batch, table):
    ids = batch.astype(jnp.int32)
    pe = _pe()
    fused, idx2 = _fuse(table, pe, ids[:SENT_SC].reshape(NW, TPW))
    out_sc = _sc_gather(fused.reshape(MAX_LEN * VOCAB, D_MODEL), idx2)
    out_tc = _tc_half(ids[SENT_SC:], table, pe)
    return jnp.concatenate(
        [out_sc.reshape(SENT_SC, MAX_LEN, D_MODEL), out_tc], axis=0
    )


# speedup vs baseline: 1.4960x; 1.4960x over previous
import functools

import jax
import jax.numpy as jnp
from jax import lax
from jax.experimental import pallas as pl
from jax.experimental.pallas import tpu as pltpu
from jax.experimental.pallas import tpu_sc as plsc

BATCH = 1024
MAX_LEN = 200
D_MODEL = 128
VOCAB = 128
LANES = 16

NC, NS = 2, 16
NW = NC * NS
TOK = BATCH * MAX_LEN
TPW = TOK // NW
CH = 64
NCHUNK = TPW // CH
NBUF = 10
NGROUP = NCHUNK // NBUF


def _pe():
    pos = jnp.arange(MAX_LEN, dtype=jnp.float32)[:, None]
    i = jnp.arange(0, D_MODEL, 2, dtype=jnp.float32)
    div = jnp.exp(-(jnp.log(10000.0) / D_MODEL) * i)
    ang = pos * div[None, :]
    pe = jnp.zeros((MAX_LEN, D_MODEL), dtype=jnp.float32)
    pe = pe.at[:, 0::2].set(jnp.sin(ang))
    pe = pe.at[:, 1::2].set(jnp.cos(ang))
    return pe


def _fuse_body(table_ref, pe_ref, idx_ref, fused_ref, idx2_ref):
    fused_ref[...] = pe_ref[...][:, None, :] + table_ref[...][None, :, :]
    t = lax.broadcasted_iota(jnp.int32, (NW, TPW), 1)
    idx2_ref[...] = idx_ref[...] + lax.rem(t, MAX_LEN) * VOCAB


_fuse = pl.pallas_call(
    _fuse_body,
    out_shape=(
        jax.ShapeDtypeStruct((MAX_LEN, VOCAB, D_MODEL), jnp.float32),
        jax.ShapeDtypeStruct((NW, TPW), jnp.int32),
    ),
)

_mesh = plsc.VectorSubcoreMesh(core_axis_name="c", subcore_axis_name="s")


@functools.partial(
    pl.kernel,
    out_type=jax.ShapeDtypeStruct((TOK, D_MODEL), jnp.float32),
    mesh=_mesh,
    scratch_types=[
        pltpu.VMEM((TPW,), jnp.int32),
        pltpu.VMEM((NBUF, CH, D_MODEL), jnp.float32),
        pltpu.SemaphoreType.DMA((NBUF,)),
        pltpu.SemaphoreType.DMA((NBUF,)),
    ],
)
def _sc_gather(fused_hbm, idx_hbm, out_hbm, idx_v, rows_v, gsem, ssem):
    wid = lax.axis_index("s") * NC + lax.axis_index("c")
    pltpu.sync_copy(idx_hbm.at[wid], idx_v)
    base = wid * TPW

    def out_slice(c):
        off = pl.multiple_of(base + c * CH, 8)
        return out_hbm.at[pl.ds(off, CH)]

    def fire_gather(c, b):
        return pltpu.async_copy(
            fused_hbm.at[idx_v.at[pl.ds(c * CH, CH)]], rows_v.at[b], gsem.at[b]
        )

    gds = [fire_gather(b, b) for b in range(NBUF)]
    for b in range(NBUF):
        gds[b].wait()
        pltpu.async_copy(rows_v.at[b], out_slice(b), ssem.at[b])

    for g in range(1, NGROUP):
        c0 = g * NBUF
        gds = []
        for b in range(NBUF):
            pltpu.make_async_copy(rows_v.at[b], out_slice(c0 + b), ssem.at[b]).wait()
            gds.append(fire_gather(c0 + b, b))
        for b in range(NBUF):
            gds[b].wait()
            pltpu.async_copy(rows_v.at[b], out_slice(c0 + b), ssem.at[b])

    for b in range(NBUF):
        pltpu.make_async_copy(
            rows_v.at[b], out_slice((NGROUP - 1) * NBUF + b), ssem.at[b]
        ).wait()


def kernel(batch, table):
    idx = batch.astype(jnp.int32).reshape(NW, TPW)
    fused, idx2 = _fuse(table, _pe(), idx)
    out = _sc_gather(fused.reshape(MAX_LEN * VOCAB, D_MODEL), idx2)
    return out.reshape(BATCH, MAX_LEN, D_MODEL)


# speedup vs baseline: 1.5218x; 1.0172x over previous
import functools

import jax
import jax.numpy as jnp
from jax import lax
from jax.experimental import pallas as pl
from jax.experimental.pallas import tpu as pltpu
from jax.experimental.pallas import tpu_sc as plsc

BATCH = 1024
MAX_LEN = 200
D_MODEL = 128
VOCAB = 128
LANES = 16

NC, NS = 2, 16
NW = NC * NS
TOK = BATCH * MAX_LEN
TPW = TOK // NW
CH = 80
NCHUNK = TPW // CH
NBUF = 8
NGROUP = NCHUNK // NBUF


def _pe():
    pos = jnp.arange(MAX_LEN, dtype=jnp.float32)[:, None]
    i = jnp.arange(0, D_MODEL, 2, dtype=jnp.float32)
    div = jnp.exp(-(jnp.log(10000.0) / D_MODEL) * i)
    ang = pos * div[None, :]
    pe = jnp.zeros((MAX_LEN, D_MODEL), dtype=jnp.float32)
    pe = pe.at[:, 0::2].set(jnp.sin(ang))
    pe = pe.at[:, 1::2].set(jnp.cos(ang))
    return pe


def _fuse_body(table_ref, pe_ref, idx_ref, fused_ref, idx2_ref):
    fused_ref[...] = pe_ref[...][:, None, :] + table_ref[...][None, :, :]
    t = lax.broadcasted_iota(jnp.int32, (NW, TPW), 1)
    idx2_ref[...] = idx_ref[...] + lax.rem(t, MAX_LEN) * VOCAB


_fuse = pl.pallas_call(
    _fuse_body,
    out_shape=(
        jax.ShapeDtypeStruct((MAX_LEN, VOCAB, D_MODEL), jnp.float32),
        jax.ShapeDtypeStruct((NW, TPW), jnp.int32),
    ),
)

_mesh = plsc.VectorSubcoreMesh(core_axis_name="c", subcore_axis_name="s")


@functools.partial(
    pl.kernel,
    out_type=jax.ShapeDtypeStruct((TOK, D_MODEL), jnp.float32),
    mesh=_mesh,
    scratch_types=[
        pltpu.VMEM((TPW,), jnp.int32),
        pltpu.VMEM((NBUF, CH, D_MODEL), jnp.float32),
        pltpu.SemaphoreType.DMA((NBUF,)),
        pltpu.SemaphoreType.DMA((NBUF,)),
    ],
)
def _sc_gather(fused_hbm, idx_hbm, out_hbm, idx_v, rows_v, gsem, ssem):
    wid = lax.axis_index("s") * NC + lax.axis_index("c")
    pltpu.sync_copy(idx_hbm.at[wid], idx_v)
    base = wid * TPW

    def out_slice(c):
        off = pl.multiple_of(base + c * CH, 8)
        return out_hbm.at[pl.ds(off, CH)]

    def fire_gather(c, b):
        return pltpu.async_copy(
            fused_hbm.at[idx_v.at[pl.ds(c * CH, CH)]], rows_v.at[b], gsem.at[b]
        )

    gds = [fire_gather(b, b) for b in range(NBUF)]
    for b in range(NBUF):
        gds[b].wait()
        pltpu.async_copy(rows_v.at[b], out_slice(b), ssem.at[b])

    @pl.loop(1, NGROUP)
    def group_body(g):
        c0 = g * NBUF
        gds = []
        for b in range(NBUF):
            pltpu.make_async_copy(rows_v.at[b], out_slice(c0 + b), ssem.at[b]).wait()
            gds.append(fire_gather(c0 + b, b))
        for b in range(NBUF):
            gds[b].wait()
            pltpu.async_copy(rows_v.at[b], out_slice(c0 + b), ssem.at[b])

    for b in range(NBUF):
        pltpu.make_async_copy(
            rows_v.at[b], out_slice((NGROUP - 1) * NBUF + b), ssem.at[b]
        ).wait()


def kernel(batch, table):
    idx = batch.astype(jnp.int32).reshape(NW, TPW)
    fused, idx2 = _fuse(table, _pe(), idx)
    out = _sc_gather(fused.reshape(MAX_LEN * VOCAB, D_MODEL), idx2)
    return out.reshape(BATCH, MAX_LEN, D_MODEL)


# speedup vs baseline: 1.5345x; 1.0084x over previous
import functools

import jax
import jax.numpy as jnp
from jax import lax
from jax.experimental import pallas as pl
from jax.experimental.pallas import tpu as pltpu
from jax.experimental.pallas import tpu_sc as plsc

BATCH = 1024
MAX_LEN = 200
D_MODEL = 128
VOCAB = 128

NC, NS = 2, 16
NW = NC * NS
TOK = BATCH * MAX_LEN
TPW = TOK // NW
CH = 64
NCHUNK = TPW // CH
NBUF = 10
NGROUP = NCHUNK // NBUF


def _pe():
    pos = jnp.arange(MAX_LEN, dtype=jnp.float32)[:, None]
    i = jnp.arange(0, D_MODEL, 2, dtype=jnp.float32)
    div = jnp.exp(-(jnp.log(10000.0) / D_MODEL) * i)
    ang = pos * div[None, :]
    pe = jnp.zeros((MAX_LEN, D_MODEL), dtype=jnp.float32)
    pe = pe.at[:, 0::2].set(jnp.sin(ang))
    pe = pe.at[:, 1::2].set(jnp.cos(ang))
    return pe


def _fuse_body(table_ref, pe_ref, idx_ref, fused_ref, idx2_ref):
    fused_ref[...] = pe_ref[...][:, None, :] + table_ref[...][None, :, :]
    t = lax.broadcasted_iota(jnp.int32, (NW, TPW), 1)
    idx2_ref[...] = idx_ref[...] + lax.rem(t, MAX_LEN) * VOCAB


_fuse = pl.pallas_call(
    _fuse_body,
    out_shape=(
        jax.ShapeDtypeStruct((MAX_LEN, VOCAB, D_MODEL), jnp.float32),
        jax.ShapeDtypeStruct((NW, TPW), jnp.int32),
    ),
)

_mesh = plsc.VectorSubcoreMesh(core_axis_name="c", subcore_axis_name="s")


@functools.partial(
    pl.kernel,
    out_type=jax.ShapeDtypeStruct((TOK, D_MODEL), jnp.float32),
    mesh=_mesh,
    scratch_types=[
        pltpu.VMEM((TPW,), jnp.int32),
        pltpu.VMEM((NBUF, CH, D_MODEL), jnp.float32),
        pltpu.SemaphoreType.DMA((NBUF,)),
        pltpu.SemaphoreType.DMA((NBUF,)),
    ],
)
def _sc_gather(fused_hbm, idx_hbm, out_hbm, idx_v, rows_v, gsem, ssem):
    wid = lax.axis_index("s") * NC + lax.axis_index("c")
    pltpu.sync_copy(idx_hbm.at[wid], idx_v)
    base = wid * TPW

    def out_slice(c):
        off = pl.multiple_of(base + c * CH, 8)
        return out_hbm.at[pl.ds(off, CH)]

    def fire_gather(c, b):
        return pltpu.async_copy(
            fused_hbm.at[idx_v.at[pl.ds(c * CH, CH)]], rows_v.at[b], gsem.at[b]
        )

    gds = [fire_gather(b, b) for b in range(NBUF)]
    for b in range(NBUF):
        gds[b].wait()
        pltpu.async_copy(rows_v.at[b], out_slice(b), ssem.at[b])

    @pl.loop(1, NGROUP)
    def group_body(g):
        c0 = g * NBUF
        gds = []
        for b in range(NBUF):
            pltpu.make_async_copy(rows_v.at[b], out_slice(c0 + b), ssem.at[b]).wait()
            gds.append(fire_gather(c0 + b, b))
        for b in range(NBUF):
            gds[b].wait()
            pltpu.async_copy(rows_v.at[b], out_slice(c0 + b), ssem.at[b])

    for b in range(NBUF):
        pltpu.make_async_copy(
            rows_v.at[b], out_slice((NGROUP - 1) * NBUF + b), ssem.at[b]
        ).wait()


def kernel(batch, table):
    idx = batch.astype(jnp.int32).reshape(NW, TPW)
    fused, idx2 = _fuse(table, _pe(), idx)
    out = _sc_gather(fused.reshape(MAX_LEN * VOCAB, D_MODEL), idx2)
    return out.reshape(BATCH, MAX_LEN, D_MODEL)
